# baseline (speedup 1.0000x reference)
import functools

import jax
import jax.numpy as jnp
from jax import lax
from jax.experimental import pallas as pl
from jax.experimental.pallas import tpu as pltpu
from jax.experimental.pallas import tpu_sc as plsc


@functools.lru_cache(maxsize=None)
def _make_gather(B, DU, DM):
    info = plsc.get_sparse_core_info()
    NC, NS = info.num_cores, info.num_subcores
    NW = NC * NS
    b_per_w = B // NW
    CH = 16
    n_ch = b_per_w // CH
    mesh = plsc.VectorSubcoreMesh(core_axis_name="c", subcore_axis_name="s")

    @functools.partial(
        pl.kernel,
        mesh=mesh,
        out_type=(jax.ShapeDtypeStruct((B, DU), jnp.float32),
                  jax.ShapeDtypeStruct((B, DM), jnp.float32)),
        scratch_types=[
            pltpu.VMEM((b_per_w,), jnp.int32),
            pltpu.VMEM((b_per_w,), jnp.int32),
            pltpu.SemaphoreType.DMA,
        ],
    )
    def gather(uid_hbm, mid_hbm, utab_hbm, mtab_hbm, out_u, out_m,
               uidx_v, midx_v, sem):
        wid = lax.axis_index("s") * NC + lax.axis_index("c")
        base = wid * b_per_w
        pltpu.sync_copy(uid_hbm.at[pl.ds(base, b_per_w)], uidx_v)
        pltpu.sync_copy(mid_hbm.at[pl.ds(base, b_per_w)], midx_v)

        def fire(c):
            uvec = uidx_v[pl.ds(c * CH, CH)]
            mvec = midx_v[pl.ds(c * CH, CH)]
            for j in range(CH):
                i = base + c * CH + j
                pltpu.async_copy(utab_hbm.at[uvec[j]], out_u.at[i], sem)
                pltpu.async_copy(mtab_hbm.at[mvec[j]], out_m.at[i], sem)

        def drain(c):
            pltpu.make_async_copy(
                utab_hbm.at[pl.ds(0, CH)],
                out_u.at[pl.ds(base + c * CH, CH)], sem).wait()
            pltpu.make_async_copy(
                mtab_hbm.at[pl.ds(0, CH)],
                out_m.at[pl.ds(base + c * CH, CH)], sem).wait()

        fire(0)

        def body(c, _):
            fire(c)
            drain(c - 1)
            return _

        lax.fori_loop(1, n_ch, body, None, unroll=False)
        drain(n_ch - 1)

    return gather, NW, n_ch, CH


def _mlp_body(ue_ref, me_ref, Wu_ref, bu_ref, Wm_ref, bm_ref,
              W1u_ref, W1m_ref, b1_ref, W2_ref, b2_ref, W3_ref, b3_ref,
              out_ref):
    dn = (((1,), (1,)), ((), ()))
    f32 = jnp.float32
    u = lax.dot_general(ue_ref[...], Wu_ref[...], dn,
                        preferred_element_type=f32) + bu_ref[...]
    m = lax.dot_general(me_ref[...], Wm_ref[...], dn,
                        preferred_element_type=f32) + bm_ref[...]
    x1 = (lax.dot_general(u, W1u_ref[...], dn, preferred_element_type=f32)
          + lax.dot_general(m, W1m_ref[...], dn, preferred_element_type=f32)
          + b1_ref[...])
    h = lax.dot_general(jnp.maximum(x1, 0.0), W2_ref[...], dn,
                        preferred_element_type=f32) + b2_ref[...] + x1
    o = jnp.sum(jnp.maximum(h, 0.0) * W3_ref[...], axis=1, keepdims=True)
    out_ref[...] = jax.nn.sigmoid(o + b3_ref[...])


@functools.lru_cache(maxsize=None)
def _make_mlp(B, DU, DM, H):
    BLK = 2048
    grid = (B // BLK,)
    full = lambda shape: pl.BlockSpec(shape, lambda i: (0,) * len(shape))
    return pl.pallas_call(
        _mlp_body,
        grid=grid,
        in_specs=[
            pl.BlockSpec((BLK, DU), lambda i: (i, 0)),
            pl.BlockSpec((BLK, DM), lambda i: (i, 0)),
            full((DU, DU)), full((1, DU)),
            full((DM, DM)), full((1, DM)),
            full((H, DU)), full((H, DM)), full((1, H)),
            full((H, H)), full((1, H)),
            full((1, H)), full((1, 1)),
        ],
        out_specs=pl.BlockSpec((BLK, 1), lambda i: (i, 0)),
        out_shape=jax.ShapeDtypeStruct((B, 1), jnp.float32),
        compiler_params=pltpu.CompilerParams(
            dimension_semantics=("arbitrary",),
        ),
    )


def kernel(user_id, movie_id, user_table, movie_table,
           W_u, b_u, W_m, b_m, W1, b1, W2, b2, W3, b3):
    B = user_id.shape[0]
    DU = user_table.shape[1]
    DM = movie_table.shape[1]
    H = W1.shape[0]

    gather, NW, n_ch, CH = _make_gather(B, DU, DM)
    ue, me = gather(user_id.astype(jnp.int32), movie_id.astype(jnp.int32),
                    user_table, movie_table)

    mlp = _make_mlp(B, DU, DM, H)
    return mlp(ue, me,
               W_u, b_u.reshape(1, DU),
               W_m, b_m.reshape(1, DM),
               W1[:, :DU], W1[:, DU:], b1.reshape(1, H),
               W2, b2.reshape(1, H),
               W3, b3.reshape(1, 1))


# speedup vs baseline: 1.7688x; 1.7688x over previous
import functools

import jax
import jax.numpy as jnp
from jax import lax
from jax.experimental import pallas as pl
from jax.experimental.pallas import tpu as pltpu
from jax.experimental.pallas import tpu_sc as plsc


@functools.lru_cache(maxsize=None)
def _make_sc_gather(B, D):
    info = plsc.get_sparse_core_info()
    NC, NS = info.num_cores, info.num_subcores
    NW = NC * NS
    b_per_w = B // NW
    CH = 128
    n_ch = b_per_w // CH
    mesh = plsc.VectorSubcoreMesh(core_axis_name="c", subcore_axis_name="s")

    @functools.partial(
        pl.kernel,
        mesh=mesh,
        out_type=jax.ShapeDtypeStruct((B, D), jnp.float32),
        scratch_types=[
            pltpu.VMEM((n_ch, CH), jnp.int32),
            pltpu.VMEM((b_per_w, D), jnp.float32),
            pltpu.SemaphoreType.DMA,
        ],
        compiler_params=pltpu.CompilerParams(use_tc_tiling_on_sc=False),
    )
    def gather(mid_hbm, mtab_hbm, out_m, midx_v, mrows_v, sem):
        wid = lax.axis_index("s") * NC + lax.axis_index("c")
        base = wid * b_per_w
        pltpu.sync_copy(mid_hbm.at[wid], midx_v)
        copies = []
        for c in range(n_ch):
            copies.append(pltpu.async_copy(
                mtab_hbm.at[midx_v.at[c]], mrows_v.at[pl.ds(c * CH, CH)], sem))
        for cp in copies:
            cp.wait()
        pltpu.sync_copy(mrows_v, out_m.at[pl.ds(base, b_per_w)])

    return gather, NW, n_ch, CH


def _tc_body(uid_s, utab_ref, me_ref, Wu_ref, bu_ref, Wm_ref, bm_ref,
             W1u_ref, W1m_ref, b1_ref, W2_ref, b2_ref, W3_ref, b3_ref,
             out_ref, ubuf, sem, *, BLK, UNROLL):
    c = pl.program_id(0)

    def fire(k, _):
        for u in range(UNROLL):
            j = k * UNROLL + u
            r = uid_s[c * BLK + j]
            pltpu.make_async_copy(utab_ref.at[r], ubuf.at[j], sem).start()
        return _

    lax.fori_loop(0, BLK // UNROLL, fire, None, unroll=False)
    pltpu.make_async_copy(utab_ref.at[pl.ds(0, BLK)], ubuf, sem).wait()

    dn = (((1,), (1,)), ((), ()))
    f32 = jnp.float32
    u = lax.dot_general(ubuf[...], Wu_ref[...], dn,
                        preferred_element_type=f32) + bu_ref[...]
    m = lax.dot_general(me_ref[...], Wm_ref[...], dn,
                        preferred_element_type=f32) + bm_ref[...]
    x1 = (lax.dot_general(u, W1u_ref[...], dn, preferred_element_type=f32)
          + lax.dot_general(m, W1m_ref[...], dn, preferred_element_type=f32)
          + b1_ref[...])
    h = lax.dot_general(jnp.maximum(x1, 0.0), W2_ref[...], dn,
                        preferred_element_type=f32) + b2_ref[...] + x1
    o = jnp.sum(jnp.maximum(h, 0.0) * W3_ref[...], axis=1, keepdims=True)
    out_ref[...] = jax.nn.sigmoid(o + b3_ref[...])


@functools.lru_cache(maxsize=None)
def _make_tc(B, DU, DM, H):
    BLK = 2048
    UNROLL = 8
    grid = (B // BLK,)
    full = lambda shape: pl.BlockSpec(shape, lambda *a: (0,) * len(shape))
    grid_spec = pltpu.PrefetchScalarGridSpec(
        num_scalar_prefetch=1,
        grid=grid,
        in_specs=[
            pl.BlockSpec(memory_space=pl.ANY),
            pl.BlockSpec((BLK, DM), lambda i, uid: (i, 0)),
            full((DU, DU)), full((1, DU)),
            full((DM, DM)), full((1, DM)),
            full((H, DU)), full((H, DM)), full((1, H)),
            full((H, H)), full((1, H)),
            full((1, H)), full((1, 1)),
        ],
        out_specs=pl.BlockSpec((BLK, 1), lambda i, uid: (i, 0)),
        scratch_shapes=[
            pltpu.VMEM((BLK, DU), jnp.float32),
            pltpu.SemaphoreType.DMA,
        ],
    )
    return pl.pallas_call(
        functools.partial(_tc_body, BLK=BLK, UNROLL=UNROLL),
        grid_spec=grid_spec,
        out_shape=jax.ShapeDtypeStruct((B, 1), jnp.float32),
        compiler_params=pltpu.CompilerParams(
            dimension_semantics=("arbitrary",),
        ),
    )


def kernel(user_id, movie_id, user_table, movie_table,
           W_u, b_u, W_m, b_m, W1, b1, W2, b2, W3, b3):
    B = user_id.shape[0]
    DU = user_table.shape[1]
    DM = movie_table.shape[1]
    H = W1.shape[0]

    sc_gather, NW, n_ch, CH = _make_sc_gather(B, DM)
    mid = movie_id.astype(jnp.int32).reshape(NW, n_ch, CH)
    me = sc_gather(mid, movie_table)

    tc = _make_tc(B, DU, DM, H)
    return tc(user_id.astype(jnp.int32), user_table, me,
              W_u, b_u.reshape(1, DU),
              W_m, b_m.reshape(1, DM),
              W1[:, :DU], W1[:, DU:], b1.reshape(1, H),
              W2, b2.reshape(1, H),
              W3, b3.reshape(1, 1))


# speedup vs baseline: 2.7992x; 1.5825x over previous
import functools

import jax
import jax.numpy as jnp
from jax import lax
from jax.experimental import pallas as pl
from jax.experimental.pallas import tpu as pltpu
from jax.experimental.pallas import tpu_sc as plsc


@functools.lru_cache(maxsize=None)
def _make_user_gather(B, DU, U):
    info = plsc.get_sparse_core_info()
    NC, NS = info.num_cores, info.num_subcores
    NW = NC * NS
    b_per_w = B // NW
    G = 16
    n_grp = b_per_w // G
    NBUF = 8
    mesh = plsc.VectorSubcoreMesh(core_axis_name="c", subcore_axis_name="s")

    @functools.partial(
        pl.kernel,
        mesh=mesh,
        out_type=jax.ShapeDtypeStruct((B, DU), jnp.float32),
        scratch_types=[
            pltpu.VMEM((b_per_w,), jnp.int32),
            pltpu.VMEM((NBUF, DU, 128), jnp.float32),
            pltpu.VMEM((G, DU), jnp.float32),
        ] + [pltpu.SemaphoreType.DMA] * NBUF,
        compiler_params=pltpu.CompilerParams(needs_layout_passes=False),
    )
    def gather(uid_hbm, utabT_hbm, out_u, uidx_v, slabs, chunk, *sems):
        wid = lax.axis_index("s") * NC + lax.axis_index("c")
        base = wid * b_per_w
        pltpu.sync_copy(uid_hbm.at[pl.ds(base, b_per_w)], uidx_v)
        iotas = [lax.iota(jnp.int32, 16) + 16 * g for g in range(DU // 16)]

        def body(grp, _):
            vec = uidx_v[pl.ds(grp * G, G)]
            rs = [vec[j] for j in range(G)]
            ts = [r // 128 for r in rs]
            ls = [r - t * 128 for r, t in zip(rs, ts)]

            def fire(j):
                pltpu.async_copy(
                    utabT_hbm.at[:, pl.ds(pl.multiple_of(ts[j] * 128, 128),
                                          128)],
                    slabs.at[j % NBUF], sems[j % NBUF])

            def extract(j):
                pltpu.make_async_copy(
                    utabT_hbm.at[:, pl.ds(0, 128)], slabs.at[j % NBUF],
                    sems[j % NBUF]).wait()
                lane = jnp.broadcast_to(ls[j], (16,))
                for g in range(DU // 16):
                    vals = plsc.load_gather(slabs.at[j % NBUF],
                                            [iotas[g], lane])
                    chunk[j, pl.ds(g * 16, 16)] = vals

            for j in range(NBUF):
                fire(j)
            for j in range(G):
                extract(j)
                if j + NBUF < G:
                    fire(j + NBUF)
            pltpu.sync_copy(chunk, out_u.at[pl.ds(base + grp * G, G)])
            return _

        lax.fori_loop(0, n_grp, body, None, unroll=False)

    return gather


@functools.lru_cache(maxsize=None)
def _make_sc_gather(B, D):
    info = plsc.get_sparse_core_info()
    NC, NS = info.num_cores, info.num_subcores
    NW = NC * NS
    b_per_w = B // NW
    CH = 128
    n_ch = b_per_w // CH
    mesh = plsc.VectorSubcoreMesh(core_axis_name="c", subcore_axis_name="s")

    @functools.partial(
        pl.kernel,
        mesh=mesh,
        out_type=jax.ShapeDtypeStruct((B, D), jnp.float32),
        scratch_types=[
            pltpu.VMEM((n_ch, CH), jnp.int32),
            pltpu.VMEM((b_per_w, D), jnp.float32),
            pltpu.SemaphoreType.DMA,
        ],
        compiler_params=pltpu.CompilerParams(use_tc_tiling_on_sc=False),
    )
    def gather(mid_hbm, mtab_hbm, out_m, midx_v, mrows_v, sem):
        wid = lax.axis_index("s") * NC + lax.axis_index("c")
        base = wid * b_per_w
        pltpu.sync_copy(mid_hbm.at[wid], midx_v)
        copies = []
        for c in range(n_ch):
            copies.append(pltpu.async_copy(
                mtab_hbm.at[midx_v.at[c]], mrows_v.at[pl.ds(c * CH, CH)], sem))
        for cp in copies:
            cp.wait()
        pltpu.sync_copy(mrows_v, out_m.at[pl.ds(base, b_per_w)])

    return gather, NW, n_ch, CH


def _mlp_body(ue_ref, me_ref, Wu_ref, bu_ref, Wm_ref, bm_ref,
              W1u_ref, W1m_ref, b1_ref, W2_ref, b2_ref, W3_ref, b3_ref,
              out_ref):
    dn = (((1,), (1,)), ((), ()))
    f32 = jnp.float32
    u = lax.dot_general(ue_ref[...], Wu_ref[...], dn,
                        preferred_element_type=f32) + bu_ref[...]
    m = lax.dot_general(me_ref[...], Wm_ref[...], dn,
                        preferred_element_type=f32) + bm_ref[...]
    x1 = (lax.dot_general(u, W1u_ref[...], dn, preferred_element_type=f32)
          + lax.dot_general(m, W1m_ref[...], dn, preferred_element_type=f32)
          + b1_ref[...])
    h = lax.dot_general(jnp.maximum(x1, 0.0), W2_ref[...], dn,
                        preferred_element_type=f32) + b2_ref[...] + x1
    o = jnp.sum(jnp.maximum(h, 0.0) * W3_ref[...], axis=1, keepdims=True)
    out_ref[...] = jax.nn.sigmoid(o + b3_ref[...])


@functools.lru_cache(maxsize=None)
def _make_mlp(B, DU, DM, H):
    BLK = 2048
    grid = (B // BLK,)
    full = lambda shape: pl.BlockSpec(shape, lambda *a: (0,) * len(shape))
    return pl.pallas_call(
        _mlp_body,
        grid=grid,
        in_specs=[
            pl.BlockSpec((BLK, DU), lambda i: (i, 0)),
            pl.BlockSpec((BLK, DM), lambda i: (i, 0)),
            full((DU, DU)), full((1, DU)),
            full((DM, DM)), full((1, DM)),
            full((H, DU)), full((H, DM)), full((1, H)),
            full((H, H)), full((1, H)),
            full((1, H)), full((1, 1)),
        ],
        out_specs=pl.BlockSpec((BLK, 1), lambda i: (i, 0)),
        out_shape=jax.ShapeDtypeStruct((B, 1), jnp.float32),
        compiler_params=pltpu.CompilerParams(
            dimension_semantics=("arbitrary",),
        ),
    )


def kernel(user_id, movie_id, user_table, movie_table,
           W_u, b_u, W_m, b_m, W1, b1, W2, b2, W3, b3):
    B = user_id.shape[0]
    U, DU = user_table.shape
    DM = movie_table.shape[1]
    H = W1.shape[0]

    sc_gather, NW, n_ch, CH = _make_sc_gather(B, DM)
    mid = movie_id.astype(jnp.int32).reshape(NW, n_ch, CH)
    me = sc_gather(mid, movie_table)

    user_gather = _make_user_gather(B, DU, U)
    ue = user_gather(user_id.astype(jnp.int32), user_table.T)

    mlp = _make_mlp(B, DU, DM, H)
    return mlp(ue, me,
               W_u, b_u.reshape(1, DU),
               W_m, b_m.reshape(1, DM),
               W1[:, :DU], W1[:, DU:], b1.reshape(1, H),
               W2, b2.reshape(1, H),
               W3, b3.reshape(1, 1))


# speedup vs baseline: 2.9055x; 1.0380x over previous
import functools

import jax
import jax.numpy as jnp
from jax import lax
from jax.experimental import pallas as pl
from jax.experimental.pallas import tpu as pltpu
from jax.experimental.pallas import tpu_sc as plsc


@functools.lru_cache(maxsize=None)
def _make_user_gather(B, DU, U):
    info = plsc.get_sparse_core_info()
    NC, NS = info.num_cores, info.num_subcores
    NW = NC * NS
    b_per_w = B // NW
    G = 32
    n_grp = b_per_w // G
    NBUF = 12
    mesh = plsc.VectorSubcoreMesh(core_axis_name="c", subcore_axis_name="s")

    @functools.partial(
        pl.kernel,
        mesh=mesh,
        out_type=jax.ShapeDtypeStruct((B, DU), jnp.float32),
        scratch_types=[
            pltpu.VMEM((b_per_w,), jnp.int32),
            pltpu.VMEM((NBUF, DU, 128), jnp.float32),
            pltpu.VMEM((G, DU), jnp.float32),
        ] + [pltpu.SemaphoreType.DMA] * NBUF,
        compiler_params=pltpu.CompilerParams(needs_layout_passes=False),
    )
    def gather(uid_hbm, utabT_hbm, out_u, uidx_v, slabs, chunk, *sems):
        wid = lax.axis_index("s") * NC + lax.axis_index("c")
        base = wid * b_per_w
        pltpu.sync_copy(uid_hbm.at[pl.ds(base, b_per_w)], uidx_v)
        iotas = [lax.iota(jnp.int32, 16) + 16 * g for g in range(DU // 16)]

        def body(grp, _):
            vecs = [uidx_v[pl.ds(grp * G + 16 * v, 16)] for v in range(G // 16)]
            rs = [vecs[j // 16][j % 16] for j in range(G)]
            ts = [r // 128 for r in rs]
            ls = [r - t * 128 for r, t in zip(rs, ts)]

            def fire(j):
                pltpu.async_copy(
                    utabT_hbm.at[:, pl.ds(pl.multiple_of(ts[j] * 128, 128),
                                          128)],
                    slabs.at[j % NBUF], sems[j % NBUF])

            def extract(j):
                pltpu.make_async_copy(
                    utabT_hbm.at[:, pl.ds(0, 128)], slabs.at[j % NBUF],
                    sems[j % NBUF]).wait()
                lane = jnp.broadcast_to(ls[j], (16,))
                for g in range(DU // 16):
                    vals = plsc.load_gather(slabs.at[j % NBUF],
                                            [iotas[g], lane])
                    chunk[j, pl.ds(g * 16, 16)] = vals

            for j in range(NBUF):
                fire(j)
            for j in range(G):
                extract(j)
                if j + NBUF < G:
                    fire(j + NBUF)
            pltpu.sync_copy(chunk, out_u.at[pl.ds(base + grp * G, G)])
            return _

        lax.fori_loop(0, n_grp, body, None, unroll=False)

    return gather


@functools.lru_cache(maxsize=None)
def _make_sc_gather(B, D):
    info = plsc.get_sparse_core_info()
    NC, NS = info.num_cores, info.num_subcores
    NW = NC * NS
    b_per_w = B // NW
    CH = 128
    n_ch = b_per_w // CH
    mesh = plsc.VectorSubcoreMesh(core_axis_name="c", subcore_axis_name="s")

    @functools.partial(
        pl.kernel,
        mesh=mesh,
        out_type=jax.ShapeDtypeStruct((B, D), jnp.float32),
        scratch_types=[
            pltpu.VMEM((n_ch, CH), jnp.int32),
            pltpu.VMEM((b_per_w, D), jnp.float32),
            pltpu.SemaphoreType.DMA,
        ],
        compiler_params=pltpu.CompilerParams(use_tc_tiling_on_sc=False),
    )
    def gather(mid_hbm, mtab_hbm, out_m, midx_v, mrows_v, sem):
        wid = lax.axis_index("s") * NC + lax.axis_index("c")
        base = wid * b_per_w
        pltpu.sync_copy(mid_hbm.at[wid], midx_v)
        copies = []
        for c in range(n_ch):
            copies.append(pltpu.async_copy(
                mtab_hbm.at[midx_v.at[c]], mrows_v.at[pl.ds(c * CH, CH)], sem))
        for cp in copies:
            cp.wait()
        pltpu.sync_copy(mrows_v, out_m.at[pl.ds(base, b_per_w)])

    return gather, NW, n_ch, CH


def _mlp_body(ue_ref, me_ref, Wu_ref, bu_ref, Wm_ref, bm_ref,
              W1u_ref, W1m_ref, b1_ref, W2_ref, b2_ref, W3_ref, b3_ref,
              out_ref):
    dn = (((1,), (1,)), ((), ()))
    f32 = jnp.float32
    u = lax.dot_general(ue_ref[...], Wu_ref[...], dn,
                        preferred_element_type=f32) + bu_ref[...]
    m = lax.dot_general(me_ref[...], Wm_ref[...], dn,
                        preferred_element_type=f32) + bm_ref[...]
    x1 = (lax.dot_general(u, W1u_ref[...], dn, preferred_element_type=f32)
          + lax.dot_general(m, W1m_ref[...], dn, preferred_element_type=f32)
          + b1_ref[...])
    h = lax.dot_general(jnp.maximum(x1, 0.0), W2_ref[...], dn,
                        preferred_element_type=f32) + b2_ref[...] + x1
    o = jnp.sum(jnp.maximum(h, 0.0) * W3_ref[...], axis=1, keepdims=True)
    out_ref[...] = jax.nn.sigmoid(o + b3_ref[...])


@functools.lru_cache(maxsize=None)
def _make_mlp(B, DU, DM, H):
    BLK = 2048
    grid = (B // BLK,)
    full = lambda shape: pl.BlockSpec(shape, lambda *a: (0,) * len(shape))
    return pl.pallas_call(
        _mlp_body,
        grid=grid,
        in_specs=[
            pl.BlockSpec((BLK, DU), lambda i: (i, 0)),
            pl.BlockSpec((BLK, DM), lambda i: (i, 0)),
            full((DU, DU)), full((1, DU)),
            full((DM, DM)), full((1, DM)),
            full((H, DU)), full((H, DM)), full((1, H)),
            full((H, H)), full((1, H)),
            full((1, H)), full((1, 1)),
        ],
        out_specs=pl.BlockSpec((BLK, 1), lambda i: (i, 0)),
        out_shape=jax.ShapeDtypeStruct((B, 1), jnp.float32),
        compiler_params=pltpu.CompilerParams(
            dimension_semantics=("arbitrary",),
        ),
    )


def kernel(user_id, movie_id, user_table, movie_table,
           W_u, b_u, W_m, b_m, W1, b1, W2, b2, W3, b3):
    B = user_id.shape[0]
    U, DU = user_table.shape
    DM = movie_table.shape[1]
    H = W1.shape[0]

    user_gather = _make_user_gather(B, DU, U)
    ue = user_gather(user_id.astype(jnp.int32), user_table.T)

    sc_gather, NW, n_ch, CH = _make_sc_gather(B, DM)
    mid = movie_id.astype(jnp.int32).reshape(NW, n_ch, CH)
    me = sc_gather(mid, movie_table)

    mlp = _make_mlp(B, DU, DM, H)
    return mlp(ue, me,
               W_u, b_u.reshape(1, DU),
               W_m, b_m.reshape(1, DM),
               W1[:, :DU], W1[:, DU:], b1.reshape(1, H),
               W2, b2.reshape(1, H),
               W3, b3.reshape(1, 1))


# speedup vs baseline: 3.0239x; 1.0408x over previous
import functools

import jax
import jax.numpy as jnp
from jax import lax
from jax.experimental import pallas as pl
from jax.experimental.pallas import tpu as pltpu
from jax.experimental.pallas import tpu_sc as plsc


@functools.lru_cache(maxsize=None)
def _make_user_gather(B, DU, U):
    info = plsc.get_sparse_core_info()
    NC, NS = info.num_cores, info.num_subcores
    NW = NC * NS
    b_per_w = B // NW
    G = 32
    n_grp = b_per_w // G
    NBUF = 8
    mesh = plsc.VectorSubcoreMesh(core_axis_name="c", subcore_axis_name="s")

    @functools.partial(
        pl.kernel,
        mesh=mesh,
        out_type=jax.ShapeDtypeStruct((B, DU), jnp.float32),
        scratch_types=[
            pltpu.VMEM((b_per_w,), jnp.int32),
            pltpu.VMEM((NBUF, DU, 128), jnp.float32),
            pltpu.VMEM((G, DU), jnp.float32),
        ] + [pltpu.SemaphoreType.DMA] * NBUF,
        compiler_params=pltpu.CompilerParams(needs_layout_passes=False),
    )
    def gather(uid_hbm, utabT_hbm, out_u, uidx_v, slabs, chunk, *sems):
        wid = lax.axis_index("s") * NC + lax.axis_index("c")
        base = wid * b_per_w
        pltpu.sync_copy(uid_hbm.at[pl.ds(base, b_per_w)], uidx_v)
        iotas = [lax.iota(jnp.int32, 16) + 16 * g for g in range(DU // 16)]

        def slab_fire(t, slot):
            pltpu.async_copy(
                utabT_hbm.at[:, pl.ds(pl.multiple_of(t * 128, 128), 128)],
                slabs.at[slot], sems[slot])

        def row_vals(grp_base, j):
            vec = uidx_v[pl.ds(grp_base + (j // 16) * 16, 16)]
            r = vec[j % 16]
            t = r // 128
            return t, r - t * 128

        def body(grp, carry):
            gb = grp * G

            def extract(j):
                pltpu.make_async_copy(
                    utabT_hbm.at[:, pl.ds(0, 128)], slabs.at[j % NBUF],
                    sems[j % NBUF]).wait()
                _, l = row_vals(gb, j)
                lane = jnp.broadcast_to(l, (16,))
                for g in range(DU // 16):
                    vals = plsc.load_gather(slabs.at[j % NBUF],
                                            [iotas[g], lane])
                    chunk[j, pl.ds(g * 16, 16)] = vals

            for j in range(NBUF):
                t0, _ = row_vals(gb, j)
                slab_fire(t0, j % NBUF)
            for j in range(G):
                extract(j)
                k = j + NBUF
                if k < G:
                    tk, _ = row_vals(gb, k)
                    slab_fire(tk, k % NBUF)

            pltpu.sync_copy(chunk, out_u.at[pl.ds(base + gb, G)])
            return carry

        lax.fori_loop(0, n_grp, body, None, unroll=False)

    return gather


@functools.lru_cache(maxsize=None)
def _make_sc_gather(B, D):
    info = plsc.get_sparse_core_info()
    NC, NS = info.num_cores, info.num_subcores
    NW = NC * NS
    b_per_w = B // NW
    CH = 128
    n_ch = b_per_w // CH
    mesh = plsc.VectorSubcoreMesh(core_axis_name="c", subcore_axis_name="s")

    @functools.partial(
        pl.kernel,
        mesh=mesh,
        out_type=jax.ShapeDtypeStruct((B, D), jnp.float32),
        scratch_types=[
            pltpu.VMEM((n_ch, CH), jnp.int32),
            pltpu.VMEM((b_per_w, D), jnp.float32),
            pltpu.SemaphoreType.DMA,
        ],
        compiler_params=pltpu.CompilerParams(use_tc_tiling_on_sc=False),
    )
    def gather(mid_hbm, mtab_hbm, out_m, midx_v, mrows_v, sem):
        wid = lax.axis_index("s") * NC + lax.axis_index("c")
        base = wid * b_per_w
        pltpu.sync_copy(mid_hbm.at[wid], midx_v)
        copies = []
        for c in range(n_ch):
            copies.append(pltpu.async_copy(
                mtab_hbm.at[midx_v.at[c]], mrows_v.at[pl.ds(c * CH, CH)], sem))
        for cp in copies:
            cp.wait()
        pltpu.sync_copy(mrows_v, out_m.at[pl.ds(base, b_per_w)])

    return gather, NW, n_ch, CH


def _mlp_body(ue_ref, me_ref, Wu_ref, bu_ref, Wm_ref, bm_ref,
              W1u_ref, W1m_ref, b1_ref, W2_ref, b2_ref, W3_ref, b3_ref,
              out_ref):
    dn = (((1,), (1,)), ((), ()))
    f32 = jnp.float32
    u = lax.dot_general(ue_ref[...], Wu_ref[...], dn,
                        preferred_element_type=f32) + bu_ref[...]
    m = lax.dot_general(me_ref[...], Wm_ref[...], dn,
                        preferred_element_type=f32) + bm_ref[...]
    x1 = (lax.dot_general(u, W1u_ref[...], dn, preferred_element_type=f32)
          + lax.dot_general(m, W1m_ref[...], dn, preferred_element_type=f32)
          + b1_ref[...])
    h = lax.dot_general(jnp.maximum(x1, 0.0), W2_ref[...], dn,
                        preferred_element_type=f32) + b2_ref[...] + x1
    o = jnp.sum(jnp.maximum(h, 0.0) * W3_ref[...], axis=1, keepdims=True)
    out_ref[...] = jax.nn.sigmoid(o + b3_ref[...])


@functools.lru_cache(maxsize=None)
def _make_mlp(B, DU, DM, H):
    BLK = 2048
    grid = (B // BLK,)
    full = lambda shape: pl.BlockSpec(shape, lambda *a: (0,) * len(shape))
    return pl.pallas_call(
        _mlp_body,
        grid=grid,
        in_specs=[
            pl.BlockSpec((BLK, DU), lambda i: (i, 0)),
            pl.BlockSpec((BLK, DM), lambda i: (i, 0)),
            full((DU, DU)), full((1, DU)),
            full((DM, DM)), full((1, DM)),
            full((H, DU)), full((H, DM)), full((1, H)),
            full((H, H)), full((1, H)),
            full((1, H)), full((1, 1)),
        ],
        out_specs=pl.BlockSpec((BLK, 1), lambda i: (i, 0)),
        out_shape=jax.ShapeDtypeStruct((B, 1), jnp.float32),
        compiler_params=pltpu.CompilerParams(
            dimension_semantics=("arbitrary",),
        ),
    )


def kernel(user_id, movie_id, user_table, movie_table,
           W_u, b_u, W_m, b_m, W1, b1, W2, b2, W3, b3):
    B = user_id.shape[0]
    U, DU = user_table.shape
    DM = movie_table.shape[1]
    H = W1.shape[0]

    user_gather = _make_user_gather(B, DU, U)
    ue = user_gather(user_id.astype(jnp.int32), user_table.T)

    sc_gather, NW, n_ch, CH = _make_sc_gather(B, DM)
    gate = (ue[0, 0] * 0.0).astype(jnp.int32)
    mid = (movie_id.astype(jnp.int32) + gate).reshape(NW, n_ch, CH)
    me = sc_gather(mid, movie_table)

    mlp = _make_mlp(B, DU, DM, H)
    return mlp(ue, me,
               W_u, b_u.reshape(1, DU),
               W_m, b_m.reshape(1, DM),
               W1[:, :DU], W1[:, DU:], b1.reshape(1, H),
               W2, b2.reshape(1, H),
               W3, b3.reshape(1, 1))


# speedup vs baseline: 3.1619x; 1.0456x over previous
import functools

import jax
import jax.numpy as jnp
from jax import lax
from jax.experimental import pallas as pl
from jax.experimental.pallas import tpu as pltpu
from jax.experimental.pallas import tpu_sc as plsc


@functools.lru_cache(maxsize=None)
def _make_user_gather(B, DU, U):
    info = plsc.get_sparse_core_info()
    NC, NS = info.num_cores, info.num_subcores
    NW = NC * NS
    b_per_w = B // NW
    G = 32
    n_grp = b_per_w // G
    NBUF = 8
    mesh = plsc.VectorSubcoreMesh(core_axis_name="c", subcore_axis_name="s")

    @functools.partial(
        pl.kernel,
        mesh=mesh,
        out_type=jax.ShapeDtypeStruct((B, DU), jnp.float32),
        scratch_types=[
            pltpu.VMEM((b_per_w,), jnp.int32),
            pltpu.VMEM((NBUF, DU, 128), jnp.float32),
            pltpu.VMEM((G, DU), jnp.float32),
        ] + [pltpu.SemaphoreType.DMA] * NBUF,
        compiler_params=pltpu.CompilerParams(needs_layout_passes=False),
    )
    def gather(uid_hbm, utabT_hbm, out_u, uidx_v, slabs, chunk, *sems):
        wid = lax.axis_index("s") * NC + lax.axis_index("c")
        base = wid * b_per_w
        pltpu.sync_copy(uid_hbm.at[pl.ds(base, b_per_w)], uidx_v)
        iotas = [lax.iota(jnp.int32, 16) + 16 * g for g in range(DU // 16)]

        def slab_fire(t, slot):
            pltpu.async_copy(
                utabT_hbm.at[:, pl.ds(pl.multiple_of(t * 128, 128), 128)],
                slabs.at[slot], sems[slot])

        def row_vals(grp_base, j):
            vec = uidx_v[pl.ds(grp_base + (j // 16) * 16, 16)]
            r = vec[j % 16]
            t = r // 128
            return t, r - t * 128

        for j in range(NBUF):
            t0, _l0 = row_vals(0, j)
            slab_fire(t0, j)

        def body(grp, carry):
            gb = grp * G
            nb = jnp.minimum(gb + G, (n_grp - 1) * G)

            def extract(j):
                pltpu.make_async_copy(
                    utabT_hbm.at[:, pl.ds(0, 128)], slabs.at[j % NBUF],
                    sems[j % NBUF]).wait()
                _t, l = row_vals(gb, j)
                lane = jnp.broadcast_to(l, (16,))
                for g in range(DU // 16):
                    vals = plsc.load_gather(slabs.at[j % NBUF],
                                            [iotas[g], lane])
                    chunk[j, pl.ds(g * 16, 16)] = vals

            for j in range(G):
                extract(j)
                k = j + NBUF
                if k < G:
                    tk, _lk = row_vals(gb, k)
                    slab_fire(tk, k % NBUF)
                else:
                    tn, _ln = row_vals(nb, k - G)
                    slab_fire(tn, (k - G) % NBUF)

            pltpu.sync_copy(chunk, out_u.at[pl.ds(base + gb, G)])
            return carry

        lax.fori_loop(0, n_grp, body, None, unroll=False)
        for s in range(NBUF):
            pltpu.make_async_copy(
                utabT_hbm.at[:, pl.ds(0, 128)], slabs.at[s], sems[s]).wait()

    return gather


@functools.lru_cache(maxsize=None)
def _make_sc_gather(B, D):
    info = plsc.get_sparse_core_info()
    NC, NS = info.num_cores, info.num_subcores
    NW = NC * NS
    b_per_w = B // NW
    CH = 128
    n_ch = b_per_w // CH
    mesh = plsc.VectorSubcoreMesh(core_axis_name="c", subcore_axis_name="s")

    @functools.partial(
        pl.kernel,
        mesh=mesh,
        out_type=jax.ShapeDtypeStruct((B, D), jnp.float32),
        scratch_types=[
            pltpu.VMEM((n_ch, CH), jnp.int32),
            pltpu.VMEM((b_per_w, D), jnp.float32),
            pltpu.SemaphoreType.DMA,
        ],
        compiler_params=pltpu.CompilerParams(use_tc_tiling_on_sc=False),
    )
    def gather(mid_hbm, mtab_hbm, out_m, midx_v, mrows_v, sem):
        wid = lax.axis_index("s") * NC + lax.axis_index("c")
        base = wid * b_per_w
        pltpu.sync_copy(mid_hbm.at[wid], midx_v)
        copies = []
        for c in range(n_ch):
            copies.append(pltpu.async_copy(
                mtab_hbm.at[midx_v.at[c]], mrows_v.at[pl.ds(c * CH, CH)], sem))
        for cp in copies:
            cp.wait()
        pltpu.sync_copy(mrows_v, out_m.at[pl.ds(base, b_per_w)])

    return gather, NW, n_ch, CH


def _mlp_body(ue_ref, me_ref, Wu_ref, bu_ref, Wm_ref, bm_ref,
              W1u_ref, W1m_ref, b1_ref, W2_ref, b2_ref, W3_ref, b3_ref,
              out_ref):
    dn = (((1,), (1,)), ((), ()))
    f32 = jnp.float32
    u = lax.dot_general(ue_ref[...], Wu_ref[...], dn,
                        preferred_element_type=f32) + bu_ref[...]
    m = lax.dot_general(me_ref[...], Wm_ref[...], dn,
                        preferred_element_type=f32) + bm_ref[...]
    x1 = (lax.dot_general(u, W1u_ref[...], dn, preferred_element_type=f32)
          + lax.dot_general(m, W1m_ref[...], dn, preferred_element_type=f32)
          + b1_ref[...])
    h = lax.dot_general(jnp.maximum(x1, 0.0), W2_ref[...], dn,
                        preferred_element_type=f32) + b2_ref[...] + x1
    o = jnp.sum(jnp.maximum(h, 0.0) * W3_ref[...], axis=1, keepdims=True)
    out_ref[...] = jax.nn.sigmoid(o + b3_ref[...])


@functools.lru_cache(maxsize=None)
def _make_mlp(B, DU, DM, H):
    BLK = 2048
    grid = (B // BLK,)
    full = lambda shape: pl.BlockSpec(shape, lambda *a: (0,) * len(shape))
    return pl.pallas_call(
        _mlp_body,
        grid=grid,
        in_specs=[
            pl.BlockSpec((BLK, DU), lambda i: (i, 0)),
            pl.BlockSpec((BLK, DM), lambda i: (i, 0)),
            full((DU, DU)), full((1, DU)),
            full((DM, DM)), full((1, DM)),
            full((H, DU)), full((H, DM)), full((1, H)),
            full((H, H)), full((1, H)),
            full((1, H)), full((1, 1)),
        ],
        out_specs=pl.BlockSpec((BLK, 1), lambda i: (i, 0)),
        out_shape=jax.ShapeDtypeStruct((B, 1), jnp.float32),
        compiler_params=pltpu.CompilerParams(
            dimension_semantics=("arbitrary",),
        ),
    )


def kernel(user_id, movie_id, user_table, movie_table,
           W_u, b_u, W_m, b_m, W1, b1, W2, b2, W3, b3):
    B = user_id.shape[0]
    U, DU = user_table.shape
    DM = movie_table.shape[1]
    H = W1.shape[0]

    user_gather = _make_user_gather(B, DU, U)
    ue = user_gather(user_id.astype(jnp.int32), user_table.T)

    sc_gather, NW, n_ch, CH = _make_sc_gather(B, DM)
    gate = (ue[0, 0] * 0.0).astype(jnp.int32)
    mid = (movie_id.astype(jnp.int32) + gate).reshape(NW, n_ch, CH)
    me = sc_gather(mid, movie_table)

    mlp = _make_mlp(B, DU, DM, H)
    return mlp(ue, me,
               W_u, b_u.reshape(1, DU),
               W_m, b_m.reshape(1, DM),
               W1[:, :DU], W1[:, DU:], b1.reshape(1, H),
               W2, b2.reshape(1, H),
               W3, b3.reshape(1, 1))
